# Initial kernel scaffold; baseline (speedup 1.0000x reference)
import jax
import jax.numpy as jnp
from jax.experimental import pallas as pl


def kernel(atom_fea, nbr_fea, nbr_fea_idx, crystal_atom_idx, emb_W, emb_b, c0_Wf, c0_bf, c0_g1, c0_b1, c0_g2, c0_b2, c1_Wf, c1_bf, c1_g1, c1_b1, c1_g2, c1_b2, c2_Wf, c2_bf, c2_g1, c2_b1, c2_g2, c2_b2, fc1_W, fc1_b, out_wmu, out_wrho, out_bmu, out_brho, out_weps, out_beps):
    raise NotImplementedError("write your pallas kernel here")


# speedup vs baseline: 2.3701x; 2.3701x over previous
import functools

import jax
import jax.numpy as jnp
from jax import lax
from jax.experimental import pallas as pl
from jax.experimental.pallas import tpu as pltpu
from jax.experimental.pallas import tpu_sc as plsc

N, M, ORIG, A, NBR, H, CC, P = 10000, 32, 128, 128, 16, 256, 100, 100
C2 = 2 * A
E = N * M
EPS = 1e-5
F32 = jnp.float32


def _softplus(x):
    return jnp.maximum(x, 0.0) + jnp.log(1.0 + jnp.exp(-jnp.abs(x)))


def _sigmoid(x):
    return 1.0 / (1.0 + jnp.exp(-x))


_BP = 1000


def _embed_body(x_ref, w_ref, b_ref, o_ref):
    o_ref[...] = (jnp.dot(x_ref[...], w_ref[...], preferred_element_type=F32)
                  + b_ref[...])


_embed = pl.pallas_call(
    _embed_body,
    grid=(N // _BP,),
    in_specs=[
        pl.BlockSpec((_BP, ORIG), lambda i: (i, 0)),
        pl.BlockSpec((ORIG, A), lambda i: (0, 0)),
        pl.BlockSpec((1, A), lambda i: (0, 0)),
    ],
    out_specs=pl.BlockSpec((_BP, A), lambda i: (i, 0)),
    out_shape=jax.ShapeDtypeStruct((N, A), F32),
)


def _proj_body(x_ref, w_ref, bf_ref, sp_ref, np_ref):
    y = jnp.dot(x_ref[...], w_ref[...], preferred_element_type=F32)
    sp_ref[...] = y[:, :C2] + bf_ref[...]
    np_ref[...] = y[:, C2:]


_proj = pl.pallas_call(
    _proj_body,
    grid=(N // _BP,),
    in_specs=[
        pl.BlockSpec((_BP, A), lambda i: (i, 0)),
        pl.BlockSpec((A, 2 * C2), lambda i: (0, 0)),
        pl.BlockSpec((1, C2), lambda i: (0, 0)),
    ],
    out_specs=[
        pl.BlockSpec((_BP, C2), lambda i: (i, 0)),
        pl.BlockSpec((_BP, C2), lambda i: (i, 0)),
    ],
    out_shape=[jax.ShapeDtypeStruct((N, C2), F32)] * 2,
)

_NC, _NS = 2, 16
_NW = _NC * _NS
_EPW = E // _NW
_CH = 400


def _sc_gather_body(table_hbm, idx_hbm, out_hbm, idx_v, rows_v, sem):
    wid = lax.axis_index("s") * _NC + lax.axis_index("c")
    base = wid * _EPW

    def step(j, carry):
        off = base + j * _CH
        pltpu.sync_copy(idx_hbm.at[pl.ds(off, _CH)], idx_v)
        pltpu.async_copy(table_hbm.at[idx_v], rows_v, sem).wait()
        pltpu.sync_copy(rows_v, out_hbm.at[pl.ds(off, _CH)])
        return carry

    lax.fori_loop(0, _EPW // _CH, step, 0)


_sc_gather = pl.kernel(
    _sc_gather_body,
    out_type=jax.ShapeDtypeStruct((E, C2), F32),
    mesh=plsc.VectorSubcoreMesh(core_axis_name="c", subcore_axis_name="s",
                                num_cores=_NC, num_subcores=_NS),
    scratch_types=[
        pltpu.VMEM((_CH,), jnp.int32),
        pltpu.VMEM((_CH, C2), F32),
        pltpu.SemaphoreType.DMA,
    ],
)

_BN = 200
_BE = _BN * M


def _edge_t(g_ref, sp_ref, nf_ref, we_ref):
    e = jnp.dot(nf_ref[...], we_ref[...], preferred_element_type=F32)
    return g_ref[...] + e.reshape(_BN, M, C2) + sp_ref[...][:, None, :]


def _stats_body(g_ref, sp_ref, nf_ref, we_ref, o_ref):
    t = _edge_t(g_ref, sp_ref, nf_ref, we_ref).reshape(_BE, C2)
    s = jnp.sum(t, axis=0, keepdims=True)
    ss = jnp.sum(t * t, axis=0, keepdims=True)
    part = jnp.concatenate([s, ss], axis=0)

    @pl.when(pl.program_id(0) == 0)
    def _():
        o_ref[...] = jnp.zeros_like(o_ref)

    o_ref[...] += part


_stats = pl.pallas_call(
    _stats_body,
    grid=(N // _BN,),
    in_specs=[
        pl.BlockSpec((_BN, M, C2), lambda i: (i, 0, 0)),
        pl.BlockSpec((_BN, C2), lambda i: (i, 0)),
        pl.BlockSpec((_BE, NBR), lambda i: (i, 0)),
        pl.BlockSpec((NBR, C2), lambda i: (0, 0)),
    ],
    out_specs=pl.BlockSpec((2, C2), lambda i: (0, 0)),
    out_shape=jax.ShapeDtypeStruct((2, C2), F32),
)


def _gate_body(g_ref, sp_ref, nf_ref, we_ref, st_ref, g1_ref, b1_ref,
               ns_ref, st2_ref):
    inv = 1.0 / float(E)
    mean = st_ref[0:1, :] * inv
    var = st_ref[1:2, :] * inv - mean * mean
    scale = g1_ref[...] * lax.rsqrt(var + EPS)
    shift = b1_ref[...] - mean * scale
    t = _edge_t(g_ref, sp_ref, nf_ref, we_ref)
    tn = t * scale[None] + shift[None]
    ns = jnp.sum(_sigmoid(tn[:, :, :A]) * _softplus(tn[:, :, A:]), axis=1)
    ns_ref[...] = ns
    p1 = jnp.sum(ns, axis=0, keepdims=True)
    p2 = jnp.sum(ns * ns, axis=0, keepdims=True)
    part = jnp.concatenate([p1, p2], axis=0)

    @pl.when(pl.program_id(0) == 0)
    def _():
        st2_ref[...] = jnp.zeros_like(st2_ref)

    st2_ref[...] += part


_gate = pl.pallas_call(
    _gate_body,
    grid=(N // _BN,),
    in_specs=[
        pl.BlockSpec((_BN, M, C2), lambda i: (i, 0, 0)),
        pl.BlockSpec((_BN, C2), lambda i: (i, 0)),
        pl.BlockSpec((_BE, NBR), lambda i: (i, 0)),
        pl.BlockSpec((NBR, C2), lambda i: (0, 0)),
        pl.BlockSpec((2, C2), lambda i: (0, 0)),
        pl.BlockSpec((1, C2), lambda i: (0, 0)),
        pl.BlockSpec((1, C2), lambda i: (0, 0)),
    ],
    out_specs=[
        pl.BlockSpec((_BN, A), lambda i: (i, 0)),
        pl.BlockSpec((2, A), lambda i: (0, 0)),
    ],
    out_shape=[
        jax.ShapeDtypeStruct((N, A), F32),
        jax.ShapeDtypeStruct((2, A), F32),
    ],
)


def _update_body(x_ref, ns_ref, st2_ref, g2_ref, b2_ref, o_ref):
    inv = 1.0 / float(N)
    mean = st2_ref[0:1, :] * inv
    var = st2_ref[1:2, :] * inv - mean * mean
    scale = g2_ref[...] * lax.rsqrt(var + EPS)
    shift = b2_ref[...] - mean * scale
    o_ref[...] = _softplus(x_ref[...] + ns_ref[...] * scale + shift)


_update = pl.pallas_call(
    _update_body,
    grid=(N // _BP,),
    in_specs=[
        pl.BlockSpec((_BP, A), lambda i: (i, 0)),
        pl.BlockSpec((_BP, A), lambda i: (i, 0)),
        pl.BlockSpec((2, A), lambda i: (0, 0)),
        pl.BlockSpec((1, A), lambda i: (0, 0)),
        pl.BlockSpec((1, A), lambda i: (0, 0)),
    ],
    out_specs=pl.BlockSpec((_BP, A), lambda i: (i, 0)),
    out_shape=jax.ShapeDtypeStruct((N, A), F32),
)


def _pool_body(x_ref, o_ref):
    o_ref[...] = jnp.mean(x_ref[...], axis=1, keepdims=True)


_pool = pl.pallas_call(
    _pool_body,
    grid=(CC,),
    in_specs=[pl.BlockSpec((1, P, A), lambda i: (i, 0, 0))],
    out_specs=pl.BlockSpec((1, 1, A), lambda i: (i, 0, 0)),
    out_shape=jax.ShapeDtypeStruct((CC, 1, A), F32),
)


def _head_body(pool_ref, w1_ref, b1_ref, wmu_ref, wrho_ref, weps_ref,
               bmu_ref, brho_ref, beps_ref, o_ref):
    h = _softplus(pool_ref[...])
    h = jnp.dot(h, w1_ref[...], preferred_element_type=F32) + b1_ref[...]
    h = _softplus(h)
    w = wmu_ref[...] + _softplus(wrho_ref[...]) * weps_ref[...]
    b = bmu_ref[...] + _softplus(brho_ref[...]) * beps_ref[...]
    o_ref[...] = jnp.sum(h * w, axis=1, keepdims=True) + b


_head = pl.pallas_call(
    _head_body,
    out_shape=jax.ShapeDtypeStruct((CC, 1), F32),
)


def kernel(atom_fea, nbr_fea, nbr_fea_idx, crystal_atom_idx, emb_W, emb_b,
           c0_Wf, c0_bf, c0_g1, c0_b1, c0_g2, c0_b2,
           c1_Wf, c1_bf, c1_g1, c1_b1, c1_g2, c1_b2,
           c2_Wf, c2_bf, c2_g1, c2_b1, c2_g2, c2_b2,
           fc1_W, fc1_b, out_wmu, out_wrho, out_bmu, out_brho,
           out_weps, out_beps):
    x = _embed(atom_fea, emb_W.T, emb_b.reshape(1, A))
    idx_flat = nbr_fea_idx.reshape(E).astype(jnp.int32)
    nf2 = nbr_fea.reshape(E, NBR)
    for Wf, bf, g1, b1, g2, b2 in (
            (c0_Wf, c0_bf, c0_g1, c0_b1, c0_g2, c0_b2),
            (c1_Wf, c1_bf, c1_g1, c1_b1, c1_g2, c1_b2),
            (c2_Wf, c2_bf, c2_g1, c2_b1, c2_g2, c2_b2)):
        wsn = jnp.concatenate([Wf[:, :A].T, Wf[:, A:2 * A].T], axis=1)
        wet = Wf[:, 2 * A:].T
        sp, npj = _proj(x, wsn, bf.reshape(1, C2))
        g3 = _sc_gather(npj, idx_flat).reshape(N, M, C2)
        st = _stats(g3, sp, nf2, wet)
        ns, st2 = _gate(g3, sp, nf2, wet, st, g1.reshape(1, C2),
                        b1.reshape(1, C2))
        x = _update(x, ns, st2, g2.reshape(1, A), b2.reshape(1, A))
    pooled = _pool(x.reshape(CC, P, A)).reshape(CC, A)
    return _head(pooled, fc1_W.T, fc1_b.reshape(1, H), out_wmu, out_wrho,
                 out_weps, out_bmu.reshape(1, 1), out_brho.reshape(1, 1),
                 out_beps.reshape(1, 1))


# speedup vs baseline: 3.1869x; 1.3446x over previous
import functools

import jax
import jax.numpy as jnp
from jax import lax
from jax.experimental import pallas as pl
from jax.experimental.pallas import tpu as pltpu
from jax.experimental.pallas import tpu_sc as plsc

N, M, ORIG, A, NBR, H, CC, P = 10000, 32, 128, 128, 16, 256, 100, 100
C2 = 2 * A
E = N * M
EPS = 1e-5
F32 = jnp.float32


def _softplus(x):
    return jnp.maximum(x, 0.0) + jnp.log(1.0 + jnp.exp(-jnp.abs(x)))


def _sigmoid(x):
    return 1.0 / (1.0 + jnp.exp(-x))


_BP = 1000


def _embed_body(x_ref, w_ref, b_ref, o_ref):
    o_ref[...] = (jnp.dot(x_ref[...], w_ref[...], preferred_element_type=F32)
                  + b_ref[...])


_embed = pl.pallas_call(
    _embed_body,
    grid=(N // _BP,),
    in_specs=[
        pl.BlockSpec((_BP, ORIG), lambda i: (i, 0)),
        pl.BlockSpec((ORIG, A), lambda i: (0, 0)),
        pl.BlockSpec((1, A), lambda i: (0, 0)),
    ],
    out_specs=pl.BlockSpec((_BP, A), lambda i: (i, 0)),
    out_shape=jax.ShapeDtypeStruct((N, A), F32),
)


def _rne_bf16_bits(x):
    b = jax.lax.bitcast_convert_type(x, jnp.int32)
    return jax.lax.shift_right_logical(
        b + 0x7FFF + (jax.lax.shift_right_logical(b, 16) & 1), 16)


def _proj_body(x_ref, w_ref, bf_ref, sp_ref, np_ref):
    y = jnp.dot(x_ref[...], w_ref[...], preferred_element_type=F32)
    sp_ref[...] = y[:, :C2] + bf_ref[...]
    lo = _rne_bf16_bits(y[:, C2:C2 + A])
    hi = _rne_bf16_bits(y[:, C2 + A:])
    np_ref[...] = jax.lax.shift_left(hi, 16) | lo


_proj = pl.pallas_call(
    _proj_body,
    grid=(N // _BP,),
    in_specs=[
        pl.BlockSpec((_BP, A), lambda i: (i, 0)),
        pl.BlockSpec((A, 2 * C2), lambda i: (0, 0)),
        pl.BlockSpec((1, C2), lambda i: (0, 0)),
    ],
    out_specs=[
        pl.BlockSpec((_BP, C2), lambda i: (i, 0)),
        pl.BlockSpec((_BP, A), lambda i: (i, 0)),
    ],
    out_shape=[jax.ShapeDtypeStruct((N, C2), F32),
               jax.ShapeDtypeStruct((N, A), jnp.int32)],
)

_NC, _NS = 2, 16
_NW = _NC * _NS
_EPW = E // _NW
_CH = 1000


def _sc_gather_body(table_hbm, idx_hbm, out_hbm, idx_v, rows_v, sem):
    wid = lax.axis_index("s") * _NC + lax.axis_index("c")
    base = wid * _EPW

    def step(j, carry):
        off = base + j * _CH
        pltpu.sync_copy(idx_hbm.at[pl.ds(off, _CH)], idx_v)
        pltpu.async_copy(table_hbm.at[idx_v], rows_v, sem).wait()
        pltpu.sync_copy(rows_v, out_hbm.at[pl.ds(off, _CH)])
        return carry

    lax.fori_loop(0, _EPW // _CH, step, 0)


_sc_gather = pl.kernel(
    _sc_gather_body,
    out_type=jax.ShapeDtypeStruct((E, A), jnp.int32),
    mesh=plsc.VectorSubcoreMesh(core_axis_name="c", subcore_axis_name="s",
                                num_cores=_NC, num_subcores=_NS),
    scratch_types=[
        pltpu.VMEM((_CH,), jnp.int32),
        pltpu.VMEM((_CH, A), jnp.int32),
        pltpu.SemaphoreType.DMA,
    ],
)

_BN = 200
_BE = _BN * M


def _edge_halves(g_ref, sp_ref, nf_ref, we_ref):
    e = jnp.dot(nf_ref[...], we_ref[...],
                preferred_element_type=F32).reshape(_BN, M, C2)
    w = g_ref[...]
    glo = jax.lax.bitcast_convert_type(jax.lax.shift_left(w, 16), F32)
    ghi = jax.lax.bitcast_convert_type(w & jnp.int32(-65536), F32)
    sp = sp_ref[...]
    tf = glo + e[:, :, :A] + sp[:, None, :A]
    tc = ghi + e[:, :, A:] + sp[:, None, A:]
    return tf, tc


def _stats_body(g_ref, sp_ref, nf_ref, we_ref, o_ref):
    tf, tc = _edge_halves(g_ref, sp_ref, nf_ref, we_ref)
    tf2, tc2 = tf.reshape(_BE, A), tc.reshape(_BE, A)
    s = jnp.concatenate([jnp.sum(tf2, axis=0, keepdims=True),
                         jnp.sum(tc2, axis=0, keepdims=True)], axis=1)
    ss = jnp.concatenate([jnp.sum(tf2 * tf2, axis=0, keepdims=True),
                          jnp.sum(tc2 * tc2, axis=0, keepdims=True)], axis=1)
    part = jnp.concatenate([s, ss], axis=0)

    @pl.when(pl.program_id(0) == 0)
    def _():
        o_ref[...] = jnp.zeros_like(o_ref)

    o_ref[...] += part


_stats = pl.pallas_call(
    _stats_body,
    grid=(N // _BN,),
    in_specs=[
        pl.BlockSpec((_BN, M, A), lambda i: (i, 0, 0)),
        pl.BlockSpec((_BN, C2), lambda i: (i, 0)),
        pl.BlockSpec((_BE, NBR), lambda i: (i, 0)),
        pl.BlockSpec((NBR, C2), lambda i: (0, 0)),
    ],
    out_specs=pl.BlockSpec((2, C2), lambda i: (0, 0)),
    out_shape=jax.ShapeDtypeStruct((2, C2), F32),
)


def _gate_body(g_ref, sp_ref, nf_ref, we_ref, st_ref, g1_ref, b1_ref,
               ns_ref, st2_ref):
    inv = 1.0 / float(E)
    mean = st_ref[0:1, :] * inv
    var = st_ref[1:2, :] * inv - mean * mean
    scale = g1_ref[...] * lax.rsqrt(var + EPS)
    shift = b1_ref[...] - mean * scale
    tf, tc = _edge_halves(g_ref, sp_ref, nf_ref, we_ref)
    tnf = tf * scale[None, :, :A] + shift[None, :, :A]
    tnc = tc * scale[None, :, A:] + shift[None, :, A:]
    ns = jnp.sum(_sigmoid(tnf) * _softplus(tnc), axis=1)
    ns_ref[...] = ns
    p1 = jnp.sum(ns, axis=0, keepdims=True)
    p2 = jnp.sum(ns * ns, axis=0, keepdims=True)
    part = jnp.concatenate([p1, p2], axis=0)

    @pl.when(pl.program_id(0) == 0)
    def _():
        st2_ref[...] = jnp.zeros_like(st2_ref)

    st2_ref[...] += part


_gate = pl.pallas_call(
    _gate_body,
    grid=(N // _BN,),
    in_specs=[
        pl.BlockSpec((_BN, M, A), lambda i: (i, 0, 0)),
        pl.BlockSpec((_BN, C2), lambda i: (i, 0)),
        pl.BlockSpec((_BE, NBR), lambda i: (i, 0)),
        pl.BlockSpec((NBR, C2), lambda i: (0, 0)),
        pl.BlockSpec((2, C2), lambda i: (0, 0)),
        pl.BlockSpec((1, C2), lambda i: (0, 0)),
        pl.BlockSpec((1, C2), lambda i: (0, 0)),
    ],
    out_specs=[
        pl.BlockSpec((_BN, A), lambda i: (i, 0)),
        pl.BlockSpec((2, A), lambda i: (0, 0)),
    ],
    out_shape=[
        jax.ShapeDtypeStruct((N, A), F32),
        jax.ShapeDtypeStruct((2, A), F32),
    ],
)


def _update_body(x_ref, ns_ref, st2_ref, g2_ref, b2_ref, o_ref):
    inv = 1.0 / float(N)
    mean = st2_ref[0:1, :] * inv
    var = st2_ref[1:2, :] * inv - mean * mean
    scale = g2_ref[...] * lax.rsqrt(var + EPS)
    shift = b2_ref[...] - mean * scale
    o_ref[...] = _softplus(x_ref[...] + ns_ref[...] * scale + shift)


_update = pl.pallas_call(
    _update_body,
    grid=(N // _BP,),
    in_specs=[
        pl.BlockSpec((_BP, A), lambda i: (i, 0)),
        pl.BlockSpec((_BP, A), lambda i: (i, 0)),
        pl.BlockSpec((2, A), lambda i: (0, 0)),
        pl.BlockSpec((1, A), lambda i: (0, 0)),
        pl.BlockSpec((1, A), lambda i: (0, 0)),
    ],
    out_specs=pl.BlockSpec((_BP, A), lambda i: (i, 0)),
    out_shape=jax.ShapeDtypeStruct((N, A), F32),
)


def _pool_body(x_ref, o_ref):
    o_ref[...] = jnp.mean(x_ref[...], axis=1, keepdims=True)


_pool = pl.pallas_call(
    _pool_body,
    grid=(CC,),
    in_specs=[pl.BlockSpec((1, P, A), lambda i: (i, 0, 0))],
    out_specs=pl.BlockSpec((1, 1, A), lambda i: (i, 0, 0)),
    out_shape=jax.ShapeDtypeStruct((CC, 1, A), F32),
)


def _head_body(pool_ref, w1_ref, b1_ref, wmu_ref, wrho_ref, weps_ref,
               bmu_ref, brho_ref, beps_ref, o_ref):
    h = _softplus(pool_ref[...])
    h = jnp.dot(h, w1_ref[...], preferred_element_type=F32) + b1_ref[...]
    h = _softplus(h)
    w = wmu_ref[...] + _softplus(wrho_ref[...]) * weps_ref[...]
    b = bmu_ref[...] + _softplus(brho_ref[...]) * beps_ref[...]
    o_ref[...] = jnp.sum(h * w, axis=1, keepdims=True) + b


_head = pl.pallas_call(
    _head_body,
    out_shape=jax.ShapeDtypeStruct((CC, 1), F32),
)


def kernel(atom_fea, nbr_fea, nbr_fea_idx, crystal_atom_idx, emb_W, emb_b,
           c0_Wf, c0_bf, c0_g1, c0_b1, c0_g2, c0_b2,
           c1_Wf, c1_bf, c1_g1, c1_b1, c1_g2, c1_b2,
           c2_Wf, c2_bf, c2_g1, c2_b1, c2_g2, c2_b2,
           fc1_W, fc1_b, out_wmu, out_wrho, out_bmu, out_brho,
           out_weps, out_beps):
    x = _embed(atom_fea, emb_W.T, emb_b.reshape(1, A))
    idx_flat = nbr_fea_idx.reshape(E).astype(jnp.int32)
    nf2 = nbr_fea.reshape(E, NBR)
    for Wf, bf, g1, b1, g2, b2 in (
            (c0_Wf, c0_bf, c0_g1, c0_b1, c0_g2, c0_b2),
            (c1_Wf, c1_bf, c1_g1, c1_b1, c1_g2, c1_b2),
            (c2_Wf, c2_bf, c2_g1, c2_b1, c2_g2, c2_b2)):
        wsn = jnp.concatenate([Wf[:, :A].T, Wf[:, A:2 * A].T], axis=1)
        wet = Wf[:, 2 * A:].T
        sp, npj = _proj(x, wsn, bf.reshape(1, C2))
        g3 = _sc_gather(npj, idx_flat).reshape(N, M, A)
        st = _stats(g3, sp, nf2, wet)
        ns, st2 = _gate(g3, sp, nf2, wet, st, g1.reshape(1, C2),
                        b1.reshape(1, C2))
        x = _update(x, ns, st2, g2.reshape(1, A), b2.reshape(1, A))
    pooled = _pool(x.reshape(CC, P, A)).reshape(CC, A)
    return _head(pooled, fc1_W.T, fc1_b.reshape(1, H), out_wmu, out_wrho,
                 out_weps, out_bmu.reshape(1, 1), out_brho.reshape(1, 1),
                 out_beps.reshape(1, 1))


# speedup vs baseline: 3.2402x; 1.0167x over previous
import functools

import jax
import jax.numpy as jnp
from jax import lax
from jax.experimental import pallas as pl
from jax.experimental.pallas import tpu as pltpu
from jax.experimental.pallas import tpu_sc as plsc

N, M, ORIG, A, NBR, H, CC, P = 10000, 32, 128, 128, 16, 256, 100, 100
C2 = 2 * A
E = N * M
EPS = 1e-5
F32 = jnp.float32


def _softplus(x):
    return jnp.maximum(x, 0.0) + jnp.log(1.0 + jnp.exp(-jnp.abs(x)))


def _sigmoid(x):
    return 1.0 / (1.0 + jnp.exp(-x))


_BP = 1000


def _embed_body(x_ref, w_ref, b_ref, o_ref):
    o_ref[...] = (jnp.dot(x_ref[...], w_ref[...], preferred_element_type=F32)
                  + b_ref[...])


_embed = pl.pallas_call(
    _embed_body,
    grid=(N // _BP,),
    in_specs=[
        pl.BlockSpec((_BP, ORIG), lambda i: (i, 0)),
        pl.BlockSpec((ORIG, A), lambda i: (0, 0)),
        pl.BlockSpec((1, A), lambda i: (0, 0)),
    ],
    out_specs=pl.BlockSpec((_BP, A), lambda i: (i, 0)),
    out_shape=jax.ShapeDtypeStruct((N, A), F32),
)


def _rne_bf16_bits(x):
    b = jax.lax.bitcast_convert_type(x, jnp.int32)
    return jax.lax.shift_right_logical(
        b + 0x7FFF + (jax.lax.shift_right_logical(b, 16) & 1), 16)


def _proj_body(x_ref, w_ref, bf_ref, sp_ref, np_ref):
    y = jnp.dot(x_ref[...], w_ref[...], preferred_element_type=F32)
    sp_ref[...] = y[:, :C2] + bf_ref[...]
    lo = _rne_bf16_bits(y[:, C2:C2 + A])
    hi = _rne_bf16_bits(y[:, C2 + A:])
    np_ref[...] = jax.lax.shift_left(hi, 16) | lo


_proj = pl.pallas_call(
    _proj_body,
    grid=(N // _BP,),
    in_specs=[
        pl.BlockSpec((_BP, A), lambda i: (i, 0)),
        pl.BlockSpec((A, 2 * C2), lambda i: (0, 0)),
        pl.BlockSpec((1, C2), lambda i: (0, 0)),
    ],
    out_specs=[
        pl.BlockSpec((_BP, C2), lambda i: (i, 0)),
        pl.BlockSpec((_BP, A), lambda i: (i, 0)),
    ],
    out_shape=[jax.ShapeDtypeStruct((N, C2), F32),
               jax.ShapeDtypeStruct((N, A), jnp.int32)],
)

_NC, _NS = 2, 16
_NW = _NC * _NS
_KC = 2
_E2 = E // _KC
_RPW = _E2 // _NW
_CH = 200
_ITERS = _RPW // _CH


def _sc_gather_body(table_hbm, idx_hbm, out_hbm,
                    i0, i1, r0, r1, sg0, sg1, ss0, ss1):
    wid = lax.axis_index("s") * _NC + lax.axis_index("c")
    base = wid * _RPW
    idxb, rowb = (i0, i1), (r0, r1)
    gsem, ssem = (sg0, sg1), (ss0, ss1)
    gd = [None, None]
    sd = [None, None]
    pltpu.sync_copy(idx_hbm.at[pl.ds(base, _CH)], i0)
    gd[0] = pltpu.async_copy(table_hbm.at[i0], r0, sg0)
    for j in range(_ITERS):
        b, nb = j % 2, (j + 1) % 2
        if j + 1 < _ITERS:
            if sd[nb] is not None:
                sd[nb].wait()
            pltpu.sync_copy(idx_hbm.at[pl.ds(base + (j + 1) * _CH, _CH)],
                            idxb[nb])
            gd[nb] = pltpu.async_copy(table_hbm.at[idxb[nb]], rowb[nb],
                                      gsem[nb])
        gd[b].wait()
        sd[b] = pltpu.async_copy(rowb[b],
                                 out_hbm.at[pl.ds(base + j * _CH, _CH)],
                                 ssem[b])
    sd[0].wait()
    sd[1].wait()


_sc_gather = pl.kernel(
    _sc_gather_body,
    out_type=jax.ShapeDtypeStruct((_E2, A), jnp.int32),
    mesh=plsc.VectorSubcoreMesh(core_axis_name="c", subcore_axis_name="s",
                                num_cores=_NC, num_subcores=_NS),
    scratch_types=[
        pltpu.VMEM((_CH,), jnp.int32),
        pltpu.VMEM((_CH,), jnp.int32),
        pltpu.VMEM((_CH, A), jnp.int32),
        pltpu.VMEM((_CH, A), jnp.int32),
        pltpu.SemaphoreType.DMA,
        pltpu.SemaphoreType.DMA,
        pltpu.SemaphoreType.DMA,
        pltpu.SemaphoreType.DMA,
    ],
)

_BN = 200
_BE = _BN * M


def _edge_halves(g_ref, sp_ref, nf_ref, we_ref):
    e = jnp.dot(nf_ref[...], we_ref[...],
                preferred_element_type=F32).reshape(_BN, M, C2)
    w = g_ref[...]
    glo = jax.lax.bitcast_convert_type(jax.lax.shift_left(w, 16), F32)
    ghi = jax.lax.bitcast_convert_type(w & jnp.int32(-65536), F32)
    sp = sp_ref[...]
    tf = glo + e[:, :, :A] + sp[:, None, :A]
    tc = ghi + e[:, :, A:] + sp[:, None, A:]
    return tf, tc


def _stats_body(g_ref, sp_ref, nf_ref, we_ref, o_ref):
    tf, tc = _edge_halves(g_ref, sp_ref, nf_ref, we_ref)
    tf2, tc2 = tf.reshape(_BE, A), tc.reshape(_BE, A)
    s = jnp.concatenate([jnp.sum(tf2, axis=0, keepdims=True),
                         jnp.sum(tc2, axis=0, keepdims=True)], axis=1)
    ss = jnp.concatenate([jnp.sum(tf2 * tf2, axis=0, keepdims=True),
                          jnp.sum(tc2 * tc2, axis=0, keepdims=True)], axis=1)
    part = jnp.concatenate([s, ss], axis=0)

    @pl.when(pl.program_id(0) == 0)
    def _():
        o_ref[...] = jnp.zeros_like(o_ref)

    o_ref[...] += part


_NK = N // _KC
_GS = _NK // _BN


def _make_stats(k):
    off = k * _GS
    return pl.pallas_call(
        _stats_body,
        grid=(_GS,),
        in_specs=[
            pl.BlockSpec((_BN, M, A), lambda i: (i, 0, 0)),
            pl.BlockSpec((_BN, C2), lambda i, o=off: (i + o, 0)),
            pl.BlockSpec((_BE, NBR), lambda i, o=off: (i + o, 0)),
            pl.BlockSpec((NBR, C2), lambda i: (0, 0)),
        ],
        out_specs=pl.BlockSpec((2, C2), lambda i: (0, 0)),
        out_shape=jax.ShapeDtypeStruct((2, C2), F32),
    )


_stats_k = [_make_stats(k) for k in range(_KC)]

_LOG2E = 1.4426950408889634
_LN2 = 0.6931471805599453


def _gate_body(g_ref, sp_ref, nf_ref, we_ref, st_ref, g1_ref, b1_ref,
               ns_ref, st2_ref):
    inv = 1.0 / float(E)
    mean = st_ref[0:1, :] * inv
    var = st_ref[1:2, :] * inv - mean * mean
    scale = g1_ref[...] * lax.rsqrt(var + EPS)
    shift = b1_ref[...] - mean * scale
    fold = jnp.concatenate([jnp.full((1, A), -_LOG2E, F32),
                            jnp.full((1, A), _LOG2E, F32)], axis=1)
    fs = scale * fold
    fu = shift * fold
    wes = we_ref[...] * fs
    sps = sp_ref[...] * fs + fu
    e = jnp.dot(nf_ref[...], wes, preferred_element_type=F32
                ).reshape(_BN, M, C2)
    w = g_ref[...]
    glo = jax.lax.bitcast_convert_type(jax.lax.shift_left(w, 16), F32)
    ghi = jax.lax.bitcast_convert_type(w & jnp.int32(-65536), F32)
    vf = glo * fs[None, :, :A] + (e[:, :, :A] + sps[:, None, :A])
    uc = ghi * fs[None, :, A:] + (e[:, :, A:] + sps[:, None, A:])
    q = jnp.maximum(uc, 0.0) + jnp.log2(1.0 + jnp.exp2(-jnp.abs(uc)))
    prod = (_LN2 * q) / (1.0 + jnp.exp2(vf))
    ns = jnp.sum(prod, axis=1)
    ns_ref[...] = ns
    p1 = jnp.sum(ns, axis=0, keepdims=True)
    p2 = jnp.sum(ns * ns, axis=0, keepdims=True)
    part = jnp.concatenate([p1, p2], axis=0)

    @pl.when(pl.program_id(0) == 0)
    def _():
        st2_ref[...] = jnp.zeros_like(st2_ref)

    st2_ref[...] += part


def _make_gate(k):
    off = k * _GS
    return pl.pallas_call(
        _gate_body,
        grid=(_GS,),
        in_specs=[
            pl.BlockSpec((_BN, M, A), lambda i: (i, 0, 0)),
            pl.BlockSpec((_BN, C2), lambda i, o=off: (i + o, 0)),
            pl.BlockSpec((_BE, NBR), lambda i, o=off: (i + o, 0)),
            pl.BlockSpec((NBR, C2), lambda i: (0, 0)),
            pl.BlockSpec((2, C2), lambda i: (0, 0)),
            pl.BlockSpec((1, C2), lambda i: (0, 0)),
            pl.BlockSpec((1, C2), lambda i: (0, 0)),
        ],
        out_specs=[
            pl.BlockSpec((_BN, A), lambda i: (i, 0)),
            pl.BlockSpec((2, A), lambda i: (0, 0)),
        ],
        out_shape=[
            jax.ShapeDtypeStruct((_NK, A), F32),
            jax.ShapeDtypeStruct((2, A), F32),
        ],
    )


_gate_k = [_make_gate(k) for k in range(_KC)]


_HB = N // _BP // _KC


def _update_body(x_ref, ns1_ref, ns2_ref, st2_ref, g2_ref, b2_ref, o_ref):
    inv = 1.0 / float(N)
    mean = st2_ref[0:1, :] * inv
    var = st2_ref[1:2, :] * inv - mean * mean
    scale = g2_ref[...] * lax.rsqrt(var + EPS)
    shift = b2_ref[...] - mean * scale
    ns = jnp.where(pl.program_id(0) < _HB, ns1_ref[...], ns2_ref[...])
    o_ref[...] = _softplus(x_ref[...] + ns * scale + shift)


_update = pl.pallas_call(
    _update_body,
    grid=(N // _BP,),
    in_specs=[
        pl.BlockSpec((_BP, A), lambda i: (i, 0)),
        pl.BlockSpec((_BP, A), lambda i: (jnp.minimum(i, _HB - 1), 0)),
        pl.BlockSpec((_BP, A), lambda i: (jnp.maximum(i - _HB, 0), 0)),
        pl.BlockSpec((2, A), lambda i: (0, 0)),
        pl.BlockSpec((1, A), lambda i: (0, 0)),
        pl.BlockSpec((1, A), lambda i: (0, 0)),
    ],
    out_specs=pl.BlockSpec((_BP, A), lambda i: (i, 0)),
    out_shape=jax.ShapeDtypeStruct((N, A), F32),
)


def _pool_body(x_ref, o_ref):
    o_ref[...] = jnp.mean(x_ref[...], axis=1, keepdims=True)


_pool = pl.pallas_call(
    _pool_body,
    grid=(CC,),
    in_specs=[pl.BlockSpec((1, P, A), lambda i: (i, 0, 0))],
    out_specs=pl.BlockSpec((1, 1, A), lambda i: (i, 0, 0)),
    out_shape=jax.ShapeDtypeStruct((CC, 1, A), F32),
)


def _head_body(pool_ref, w1_ref, b1_ref, wmu_ref, wrho_ref, weps_ref,
               bmu_ref, brho_ref, beps_ref, o_ref):
    h = _softplus(pool_ref[...])
    h = jnp.dot(h, w1_ref[...], preferred_element_type=F32) + b1_ref[...]
    h = _softplus(h)
    w = wmu_ref[...] + _softplus(wrho_ref[...]) * weps_ref[...]
    b = bmu_ref[...] + _softplus(brho_ref[...]) * beps_ref[...]
    o_ref[...] = jnp.sum(h * w, axis=1, keepdims=True) + b


_head = pl.pallas_call(
    _head_body,
    out_shape=jax.ShapeDtypeStruct((CC, 1), F32),
)


def kernel(atom_fea, nbr_fea, nbr_fea_idx, crystal_atom_idx, emb_W, emb_b,
           c0_Wf, c0_bf, c0_g1, c0_b1, c0_g2, c0_b2,
           c1_Wf, c1_bf, c1_g1, c1_b1, c1_g2, c1_b2,
           c2_Wf, c2_bf, c2_g1, c2_b1, c2_g2, c2_b2,
           fc1_W, fc1_b, out_wmu, out_wrho, out_bmu, out_brho,
           out_weps, out_beps):
    x = _embed(atom_fea, emb_W.T, emb_b.reshape(1, A))
    idx_flat = nbr_fea_idx.reshape(E).astype(jnp.int32)
    nf2 = nbr_fea.reshape(E, NBR)
    for Wf, bf, g1, b1, g2, b2 in (
            (c0_Wf, c0_bf, c0_g1, c0_b1, c0_g2, c0_b2),
            (c1_Wf, c1_bf, c1_g1, c1_b1, c1_g2, c1_b2),
            (c2_Wf, c2_bf, c2_g1, c2_b1, c2_g2, c2_b2)):
        wsn = jnp.concatenate([Wf[:, :A].T, Wf[:, A:2 * A].T], axis=1)
        wet = Wf[:, 2 * A:].T
        sp, npj = _proj(x, wsn, bf.reshape(1, C2))
        gk = [_sc_gather(npj, idx_flat[k * _E2:(k + 1) * _E2])
              .reshape(_NK, M, A) for k in range(_KC)]
        st = sum(_stats_k[k](gk[k], sp, nf2, wet) for k in range(_KC))
        g1r, b1r = g1.reshape(1, C2), b1.reshape(1, C2)
        ns_parts, st2 = [], 0
        for k in range(_KC):
            ns_k, st2_k = _gate_k[k](gk[k], sp, nf2, wet, st, g1r, b1r)
            ns_parts.append(ns_k)
            st2 = st2 + st2_k
        x = _update(x, ns_parts[0], ns_parts[1], st2,
                    g2.reshape(1, A), b2.reshape(1, A))
    pooled = _pool(x.reshape(CC, P, A)).reshape(CC, A)
    return _head(pooled, fc1_W.T, fc1_b.reshape(1, H), out_wmu, out_wrho,
                 out_weps, out_bmu.reshape(1, 1), out_brho.reshape(1, 1),
                 out_beps.reshape(1, 1))


# speedup vs baseline: 3.2618x; 1.0067x over previous
import functools

import jax
import jax.numpy as jnp
from jax import lax
from jax.experimental import pallas as pl
from jax.experimental.pallas import tpu as pltpu
from jax.experimental.pallas import tpu_sc as plsc

N, M, ORIG, A, NBR, H, CC, P = 10000, 32, 128, 128, 16, 256, 100, 100
C2 = 2 * A
E = N * M
EPS = 1e-5
F32 = jnp.float32


def _softplus(x):
    return jnp.maximum(x, 0.0) + jnp.log(1.0 + jnp.exp(-jnp.abs(x)))


def _sigmoid(x):
    return 1.0 / (1.0 + jnp.exp(-x))


_BP = 1000


def _embed_body(x_ref, w_ref, b_ref, o_ref):
    o_ref[...] = (jnp.dot(x_ref[...], w_ref[...], preferred_element_type=F32)
                  + b_ref[...])


_embed = pl.pallas_call(
    _embed_body,
    grid=(N // _BP,),
    in_specs=[
        pl.BlockSpec((_BP, ORIG), lambda i: (i, 0)),
        pl.BlockSpec((ORIG, A), lambda i: (0, 0)),
        pl.BlockSpec((1, A), lambda i: (0, 0)),
    ],
    out_specs=pl.BlockSpec((_BP, A), lambda i: (i, 0)),
    out_shape=jax.ShapeDtypeStruct((N, A), F32),
)


def _rne_bf16_bits(x):
    b = jax.lax.bitcast_convert_type(x, jnp.int32)
    return jax.lax.shift_right_logical(
        b + 0x7FFF + (jax.lax.shift_right_logical(b, 16) & 1), 16)


def _proj_body(x_ref, w_ref, bf_ref, sp_ref, np_ref):
    y = jnp.dot(x_ref[...], w_ref[...], preferred_element_type=F32)
    sp_ref[...] = y[:, :C2] + bf_ref[...]
    lo = _rne_bf16_bits(y[:, C2:C2 + A])
    hi = _rne_bf16_bits(y[:, C2 + A:])
    np_ref[...] = jax.lax.shift_left(hi, 16) | lo


_proj = pl.pallas_call(
    _proj_body,
    grid=(N // _BP,),
    in_specs=[
        pl.BlockSpec((_BP, A), lambda i: (i, 0)),
        pl.BlockSpec((A, 2 * C2), lambda i: (0, 0)),
        pl.BlockSpec((1, C2), lambda i: (0, 0)),
    ],
    out_specs=[
        pl.BlockSpec((_BP, C2), lambda i: (i, 0)),
        pl.BlockSpec((_BP, A), lambda i: (i, 0)),
    ],
    out_shape=[jax.ShapeDtypeStruct((N, C2), F32),
               jax.ShapeDtypeStruct((N, A), jnp.int32)],
)

_NC, _NS = 2, 16
_NW = _NC * _NS
_KC = 2
_E2 = E // _KC
_RPW = _E2 // _NW
_CH = 1000
_ITERS = _RPW // _CH


def _sc_gather_body(table_hbm, idx_hbm, out_hbm, idx_v, rows_v, sem):
    wid = lax.axis_index("s") * _NC + lax.axis_index("c")
    base = wid * _RPW

    def step(j, carry):
        off = base + j * _CH
        pltpu.sync_copy(idx_hbm.at[pl.ds(off, _CH)], idx_v)
        pltpu.async_copy(table_hbm.at[idx_v], rows_v, sem).wait()
        pltpu.sync_copy(rows_v, out_hbm.at[pl.ds(off, _CH)])
        return carry

    lax.fori_loop(0, _ITERS, step, 0)


_sc_gather = pl.kernel(
    _sc_gather_body,
    out_type=jax.ShapeDtypeStruct((_E2, A), jnp.int32),
    mesh=plsc.VectorSubcoreMesh(core_axis_name="c", subcore_axis_name="s",
                                num_cores=_NC, num_subcores=_NS),
    scratch_types=[
        pltpu.VMEM((_CH,), jnp.int32),
        pltpu.VMEM((_CH, A), jnp.int32),
        pltpu.SemaphoreType.DMA,
    ],
)

_BN = 200
_BE = _BN * M


def _edge_halves(g_ref, sp_ref, nf_ref, we_ref):
    e = jnp.dot(nf_ref[...], we_ref[...],
                preferred_element_type=F32).reshape(_BN, M, C2)
    w = g_ref[...]
    glo = jax.lax.bitcast_convert_type(jax.lax.shift_left(w, 16), F32)
    ghi = jax.lax.bitcast_convert_type(w & jnp.int32(-65536), F32)
    sp = sp_ref[...]
    tf = glo + e[:, :, :A] + sp[:, None, :A]
    tc = ghi + e[:, :, A:] + sp[:, None, A:]
    return tf, tc


def _stats_body(g_ref, sp_ref, nf_ref, we_ref, o_ref):
    tf, tc = _edge_halves(g_ref, sp_ref, nf_ref, we_ref)
    tf2, tc2 = tf.reshape(_BE, A), tc.reshape(_BE, A)
    s = jnp.concatenate([jnp.sum(tf2, axis=0, keepdims=True),
                         jnp.sum(tc2, axis=0, keepdims=True)], axis=1)
    ss = jnp.concatenate([jnp.sum(tf2 * tf2, axis=0, keepdims=True),
                          jnp.sum(tc2 * tc2, axis=0, keepdims=True)], axis=1)
    part = jnp.concatenate([s, ss], axis=0)

    @pl.when(pl.program_id(0) == 0)
    def _():
        o_ref[...] = jnp.zeros_like(o_ref)

    o_ref[...] += part


_NK = N // _KC
_GS = _NK // _BN


def _make_stats(k):
    off = k * _GS
    return pl.pallas_call(
        _stats_body,
        grid=(_GS,),
        in_specs=[
            pl.BlockSpec((_BN, M, A), lambda i: (i, 0, 0)),
            pl.BlockSpec((_BN, C2), lambda i, o=off: (i + o, 0)),
            pl.BlockSpec((_BE, NBR), lambda i, o=off: (i + o, 0)),
            pl.BlockSpec((NBR, C2), lambda i: (0, 0)),
        ],
        out_specs=pl.BlockSpec((2, C2), lambda i: (0, 0)),
        out_shape=jax.ShapeDtypeStruct((2, C2), F32),
    )


_stats_k = [_make_stats(k) for k in range(_KC)]

_LOG2E = 1.4426950408889634
_LN2 = 0.6931471805599453


def _gate_body(g_ref, sp_ref, nf_ref, we_ref, st_ref, g1_ref, b1_ref,
               ns_ref, st2_ref):
    inv = 1.0 / float(E)
    mean = st_ref[0:1, :] * inv
    var = st_ref[1:2, :] * inv - mean * mean
    scale = g1_ref[...] * lax.rsqrt(var + EPS)
    shift = b1_ref[...] - mean * scale
    fold = jnp.concatenate([jnp.full((1, A), -_LOG2E, F32),
                            jnp.full((1, A), _LOG2E, F32)], axis=1)
    fs = scale * fold
    fu = shift * fold
    wes = we_ref[...] * fs
    sps = sp_ref[...] * fs + fu
    e = jnp.dot(nf_ref[...], wes, preferred_element_type=F32
                ).reshape(_BN, M, C2)
    w = g_ref[...]
    glo = jax.lax.bitcast_convert_type(jax.lax.shift_left(w, 16), F32)
    ghi = jax.lax.bitcast_convert_type(w & jnp.int32(-65536), F32)
    vf = glo * fs[None, :, :A] + (e[:, :, :A] + sps[:, None, :A])
    uc = ghi * fs[None, :, A:] + (e[:, :, A:] + sps[:, None, A:])
    q = jnp.maximum(uc, 0.0) + jnp.log2(1.0 + jnp.exp2(-jnp.abs(uc)))
    prod = (_LN2 * q) / (1.0 + jnp.exp2(vf))
    ns = jnp.sum(prod, axis=1)
    ns_ref[...] = ns
    p1 = jnp.sum(ns, axis=0, keepdims=True)
    p2 = jnp.sum(ns * ns, axis=0, keepdims=True)
    part = jnp.concatenate([p1, p2], axis=0)

    @pl.when(pl.program_id(0) == 0)
    def _():
        st2_ref[...] = jnp.zeros_like(st2_ref)

    st2_ref[...] += part


def _make_gate(k):
    off = k * _GS
    return pl.pallas_call(
        _gate_body,
        grid=(_GS,),
        in_specs=[
            pl.BlockSpec((_BN, M, A), lambda i: (i, 0, 0)),
            pl.BlockSpec((_BN, C2), lambda i, o=off: (i + o, 0)),
            pl.BlockSpec((_BE, NBR), lambda i, o=off: (i + o, 0)),
            pl.BlockSpec((NBR, C2), lambda i: (0, 0)),
            pl.BlockSpec((2, C2), lambda i: (0, 0)),
            pl.BlockSpec((1, C2), lambda i: (0, 0)),
            pl.BlockSpec((1, C2), lambda i: (0, 0)),
        ],
        out_specs=[
            pl.BlockSpec((_BN, A), lambda i: (i, 0)),
            pl.BlockSpec((2, A), lambda i: (0, 0)),
        ],
        out_shape=[
            jax.ShapeDtypeStruct((_NK, A), F32),
            jax.ShapeDtypeStruct((2, A), F32),
        ],
    )


_gate_k = [_make_gate(k) for k in range(_KC)]


_HB = N // _BP // _KC


def _update_body(x_ref, ns1_ref, ns2_ref, st2_ref, g2_ref, b2_ref, o_ref):
    inv = 1.0 / float(N)
    mean = st2_ref[0:1, :] * inv
    var = st2_ref[1:2, :] * inv - mean * mean
    scale = g2_ref[...] * lax.rsqrt(var + EPS)
    shift = b2_ref[...] - mean * scale
    ns = jnp.where(pl.program_id(0) < _HB, ns1_ref[...], ns2_ref[...])
    o_ref[...] = _softplus(x_ref[...] + ns * scale + shift)


_update = pl.pallas_call(
    _update_body,
    grid=(N // _BP,),
    in_specs=[
        pl.BlockSpec((_BP, A), lambda i: (i, 0)),
        pl.BlockSpec((_BP, A), lambda i: (jnp.minimum(i, _HB - 1), 0)),
        pl.BlockSpec((_BP, A), lambda i: (jnp.maximum(i - _HB, 0), 0)),
        pl.BlockSpec((2, A), lambda i: (0, 0)),
        pl.BlockSpec((1, A), lambda i: (0, 0)),
        pl.BlockSpec((1, A), lambda i: (0, 0)),
    ],
    out_specs=pl.BlockSpec((_BP, A), lambda i: (i, 0)),
    out_shape=jax.ShapeDtypeStruct((N, A), F32),
)


def _pool_body(x_ref, o_ref):
    o_ref[...] = jnp.mean(x_ref[...], axis=1, keepdims=True)


_pool = pl.pallas_call(
    _pool_body,
    grid=(CC,),
    in_specs=[pl.BlockSpec((1, P, A), lambda i: (i, 0, 0))],
    out_specs=pl.BlockSpec((1, 1, A), lambda i: (i, 0, 0)),
    out_shape=jax.ShapeDtypeStruct((CC, 1, A), F32),
)


def _head_body(pool_ref, w1_ref, b1_ref, wmu_ref, wrho_ref, weps_ref,
               bmu_ref, brho_ref, beps_ref, o_ref):
    h = _softplus(pool_ref[...])
    h = jnp.dot(h, w1_ref[...], preferred_element_type=F32) + b1_ref[...]
    h = _softplus(h)
    w = wmu_ref[...] + _softplus(wrho_ref[...]) * weps_ref[...]
    b = bmu_ref[...] + _softplus(brho_ref[...]) * beps_ref[...]
    o_ref[...] = jnp.sum(h * w, axis=1, keepdims=True) + b


_head = pl.pallas_call(
    _head_body,
    out_shape=jax.ShapeDtypeStruct((CC, 1), F32),
)


def kernel(atom_fea, nbr_fea, nbr_fea_idx, crystal_atom_idx, emb_W, emb_b,
           c0_Wf, c0_bf, c0_g1, c0_b1, c0_g2, c0_b2,
           c1_Wf, c1_bf, c1_g1, c1_b1, c1_g2, c1_b2,
           c2_Wf, c2_bf, c2_g1, c2_b1, c2_g2, c2_b2,
           fc1_W, fc1_b, out_wmu, out_wrho, out_bmu, out_brho,
           out_weps, out_beps):
    x = _embed(atom_fea, emb_W.T, emb_b.reshape(1, A))
    idx_flat = nbr_fea_idx.reshape(E).astype(jnp.int32)
    nf2 = nbr_fea.reshape(E, NBR)
    for Wf, bf, g1, b1, g2, b2 in (
            (c0_Wf, c0_bf, c0_g1, c0_b1, c0_g2, c0_b2),
            (c1_Wf, c1_bf, c1_g1, c1_b1, c1_g2, c1_b2),
            (c2_Wf, c2_bf, c2_g1, c2_b1, c2_g2, c2_b2)):
        wsn = jnp.concatenate([Wf[:, :A].T, Wf[:, A:2 * A].T], axis=1)
        wet = Wf[:, 2 * A:].T
        sp, npj = _proj(x, wsn, bf.reshape(1, C2))
        gk = [_sc_gather(npj, idx_flat[k * _E2:(k + 1) * _E2])
              .reshape(_NK, M, A) for k in range(_KC)]
        st = sum(_stats_k[k](gk[k], sp, nf2, wet) for k in range(_KC))
        g1r, b1r = g1.reshape(1, C2), b1.reshape(1, C2)
        ns_parts, st2 = [], 0
        for k in range(_KC):
            ns_k, st2_k = _gate_k[k](gk[k], sp, nf2, wet, st, g1r, b1r)
            ns_parts.append(ns_k)
            st2 = st2 + st2_k
        x = _update(x, ns_parts[0], ns_parts[1], st2,
                    g2.reshape(1, A), b2.reshape(1, A))
    pooled = _pool(x.reshape(CC, P, A)).reshape(CC, A)
    return _head(pooled, fc1_W.T, fc1_b.reshape(1, H), out_wmu, out_wrho,
                 out_weps, out_bmu.reshape(1, 1), out_brho.reshape(1, 1),
                 out_beps.reshape(1, 1))
